# baseline (speedup 1.0000x reference)
import functools

import jax
import jax.numpy as jnp
from jax import lax
from jax.experimental import pallas as pl
from jax.experimental.pallas import tpu as pltpu
from jax.experimental.pallas import tpu_sc as plsc


def _ceil_to(a, b):
    return (a + b - 1) // b * b


_SC_INFO = plsc.get_sparse_core_info()
_NC = _SC_INFO.num_cores
_NS = _SC_INFO.num_subcores
_CH = 128


def _sc_mesh():
    return plsc.VectorSubcoreMesh(core_axis_name="c", subcore_axis_name="s")


def _sc_edge_gather(nf16, idxc2, idxr2):
    nchunks = idxc2.shape[0]
    e_pad = nchunks * _CH
    cpt = nchunks // (_NC * _NS)

    @functools.partial(
        pl.kernel,
        out_type=[jax.ShapeDtypeStruct((e_pad, 16), jnp.float32),
                  jax.ShapeDtypeStruct((e_pad, 16), jnp.float32)],
        mesh=_sc_mesh(),
        compiler_params=pltpu.CompilerParams(use_tc_tiling_on_sc=False, needs_layout_passes=False),
        scratch_types=[pltpu.VMEM((cpt, _CH), jnp.int32),
                       pltpu.VMEM((cpt, _CH), jnp.int32),
                       pltpu.VMEM((_CH, 16), jnp.float32),
                       pltpu.VMEM((_CH, 16), jnp.float32),
                       pltpu.VMEM((_CH, 16), jnp.float32),
                       pltpu.VMEM((_CH, 16), jnp.float32)]
                      + [pltpu.SemaphoreType.DMA] * 8,
    )
    def k(nf_hbm, ic_hbm, ir_hbm, gcol_hbm, grow_hbm, icv, irv,
          bc0, bc1, br0, br1, sgc0, sgc1, sgr0, sgr1, swc0, swc1, swr0, swr1):
        wid = lax.axis_index("s") * _NC + lax.axis_index("c")
        base = wid * cpt
        pltpu.sync_copy(ic_hbm.at[pl.ds(base, cpt)], icv)
        pltpu.sync_copy(ir_hbm.at[pl.ds(base, cpt)], irv)

        def gat(j, idx, buf, sem):
            pltpu.async_copy(nf_hbm.at[idx.at[j]], buf, sem)

        def gwait(buf, sem):
            pltpu.make_async_copy(nf_hbm.at[icv.at[0]], buf, sem).wait()

        def wr(j, buf, out, sem):
            pltpu.async_copy(buf, out.at[pl.ds((base + j) * _CH, _CH)], sem)

        def wwait(buf, sem):
            pltpu.make_async_copy(buf, gcol_hbm.at[pl.ds(0, _CH)], sem).wait()

        gat(0, icv, bc0, sgc0)
        gat(0, irv, br0, sgr0)
        gat(1, icv, bc1, sgc1)
        gat(1, irv, br1, sgr1)

        def body(j2, _):
            j0 = 2 * j2
            j1 = j0 + 1
            gwait(bc0, sgc0); wr(j0, bc0, gcol_hbm, swc0)
            gwait(br0, sgr0); wr(j0, br0, grow_hbm, swr0)
            gwait(bc1, sgc1); wr(j1, bc1, gcol_hbm, swc1)
            gwait(br1, sgr1); wr(j1, br1, grow_hbm, swr1)
            wwait(bc0, swc0); gat(j0 + 2, icv, bc0, sgc0)
            wwait(br0, swr0); gat(j0 + 2, irv, br0, sgr0)
            wwait(bc1, swc1); gat(j1 + 2, icv, bc1, sgc1)
            wwait(br1, swr1); gat(j1 + 2, irv, br1, sgr1)
            return _

        lax.fori_loop(0, cpt // 2 - 1, body, None)

        jl0 = cpt - 2
        jl1 = cpt - 1
        gwait(bc0, sgc0); wr(jl0, bc0, gcol_hbm, swc0)
        gwait(br0, sgr0); wr(jl0, br0, grow_hbm, swr0)
        gwait(bc1, sgc1); wr(jl1, bc1, gcol_hbm, swc1)
        gwait(br1, sgr1); wr(jl1, br1, grow_hbm, swr1)
        wwait(bc0, swc0); wwait(br0, swr0); wwait(bc1, swc1); wwait(br1, swr1)

    return k(nf16, idxc2, idxr2)


def _sc_msg_scatter(msgsplit, idxc2, n_pad):
    nchunks, _ = idxc2.shape
    w = msgsplit.shape[2]
    rps = n_pad // _NS

    ch = 64
    nch = nchunks * (_CH // ch)
    cpt = nch // _NS

    @functools.partial(
        pl.kernel,
        out_type=jax.ShapeDtypeStruct((_NC, n_pad, w), jnp.float32),
        mesh=_sc_mesh(),
        compiler_params=pltpu.CompilerParams(use_tc_tiling_on_sc=False, needs_layout_passes=False),
        scratch_types=[pltpu.VMEM((ch,), jnp.int32),
                       pltpu.VMEM((ch,), jnp.int32),
                       pltpu.VMEM((ch, w), jnp.float32),
                       pltpu.VMEM((ch, w), jnp.float32),
                       pltpu.VMEM_SHARED((n_pad, w), jnp.float32)]
                      + [pltpu.SemaphoreType.DMA] * 6,
    )
    def k(msg_hbm, ic_hbm, out_hbm, ib0, ib1, m0, m1, acc,
          sr0, sr1, ss0, ss1, si0, si1):
        c = lax.axis_index("c")
        s = lax.axis_index("s")

        def zrow(e, _):
            for q in range(w // 16):
                m0[e, pl.ds(q * 16, 16)] = jnp.zeros((16,), jnp.float32)
            return _
        lax.fori_loop(0, ch, zrow, None)

        def zcp(j, _):
            pltpu.sync_copy(m0, acc.at[pl.ds(s * rps + j * ch, ch)])
            return _
        lax.fori_loop(0, rps // ch, zcp, None)
        plsc.subcore_barrier()

        def si(j, buf, sem):
            pltpu.async_copy(ic_hbm.at[s * cpt + j], buf, sem)

        def siwait(buf, sem):
            pltpu.make_async_copy(ic_hbm.at[0], buf, sem).wait()

        def rd(j, buf, sem):
            e0 = (s * cpt + j) * ch
            pltpu.async_copy(msg_hbm.at[c, pl.ds(e0, ch)], buf, sem)

        def rdwait(buf, sem):
            pltpu.make_async_copy(msg_hbm.at[0, pl.ds(0, ch)], buf, sem).wait()

        def sca(ib, buf, sem):
            pltpu.async_copy(buf, acc.at[ib], sem, add=True)

        def scawait(buf, sem):
            pltpu.make_async_copy(buf, acc.at[ib0], sem).wait()

        si(0, ib0, si0); si(1, ib1, si1)
        rd(0, m0, sr0); rd(1, m1, sr1)

        def body(j2, _):
            j0 = 2 * j2
            j1 = j0 + 1
            rdwait(m0, sr0); siwait(ib0, si0); sca(ib0, m0, ss0)
            rdwait(m1, sr1); siwait(ib1, si1); sca(ib1, m1, ss1)
            scawait(m0, ss0); si(j0 + 2, ib0, si0); rd(j0 + 2, m0, sr0)
            scawait(m1, ss1); si(j1 + 2, ib1, si1); rd(j1 + 2, m1, sr1)
            return _
        lax.fori_loop(0, cpt // 2 - 1, body, None)

        rdwait(m0, sr0); siwait(ib0, si0); sca(ib0, m0, ss0)
        rdwait(m1, sr1); siwait(ib1, si1); sca(ib1, m1, ss1)
        scawait(m0, ss0); scawait(m1, ss1)
        plsc.subcore_barrier()

        def wout(j, _):
            r0 = s * rps + j * ch
            pltpu.sync_copy(acc.at[pl.ds(r0, ch)], m0)
            pltpu.sync_copy(m0, out_hbm.at[c, pl.ds(r0, ch)])
            return _
        lax.fori_loop(0, rps // ch, wout, None)

    return k(msgsplit, idxc2.reshape(nch, ch))


def _sc_gather_scale_scatter(g, ew2, idxr2, idxc2, n_pad, qmax=None,
                             colsplit=False, ir1=None):
    wd = g.shape[1]
    ch = _CH
    nch = idxc2.shape[0]
    cpt = nch // _NS if colsplit else nch // (_NC * _NS)
    rps = n_pad // _NS
    nq = (wd // 16) if qmax is None else qmax

    @functools.partial(
        pl.kernel,
        out_type=jax.ShapeDtypeStruct((_NC, n_pad, wd), jnp.float32),
        mesh=_sc_mesh(),
        compiler_params=pltpu.CompilerParams(use_tc_tiling_on_sc=False, needs_layout_passes=False),
        scratch_types=[pltpu.VMEM((cpt, ch), jnp.int32),
                       pltpu.VMEM((ch,), jnp.int32),
                       pltpu.VMEM((ch,), jnp.int32),
                       pltpu.VMEM((ch,), jnp.float32),
                       pltpu.VMEM((ch,), jnp.float32),
                       pltpu.VMEM((ch, wd), jnp.float32),
                       pltpu.VMEM((ch, wd), jnp.float32),
                       pltpu.VMEM_SHARED((n_pad, wd), jnp.float32)]
                      + [pltpu.SemaphoreType.DMA] * 8,
    )
    def k(g_hbm, ew_hbm, ir0_hbm, ir1_hbm, ic_hbm, out_hbm,
          icv, ib0, ib1, eb0, eb1, g0, g1, acc,
          sg0, sg1, ss0, ss1, si0, si1, se0, se1):
        c = lax.axis_index("c")
        s = lax.axis_index("s")
        base = s * cpt if colsplit else (s * _NC + c) * cpt

        def zrow(e, _):
            for q in range(wd // 16):
                g0[e, pl.ds(q * 16, 16)] = jnp.zeros((16,), jnp.float32)
            return _
        lax.fori_loop(0, ch, zrow, None)

        def zcp(j, _):
            pltpu.sync_copy(g0, acc.at[pl.ds(s * rps + j * ch, ch)])
            return _
        lax.fori_loop(0, rps // ch, zcp, None)
        plsc.subcore_barrier()

        pltpu.sync_copy(ic_hbm.at[pl.ds(base, cpt)], icv)

        if colsplit:
            def si(j, buf, sem):
                @pl.when(c == 0)
                def _a():
                    pltpu.async_copy(ir0_hbm.at[base + j], buf, sem)

                @pl.when(c != 0)
                def _b():
                    pltpu.async_copy(ir1_hbm.at[base + j], buf, sem)
        else:
            def si(j, buf, sem):
                pltpu.async_copy(ir0_hbm.at[base + j], buf, sem)

        def siwait(buf, sem):
            pltpu.make_async_copy(ir0_hbm.at[0], buf, sem).wait()

        def se(j, buf, sem):
            pltpu.async_copy(ew_hbm.at[base + j], buf, sem)

        def sewait(buf, sem):
            pltpu.make_async_copy(ew_hbm.at[0], buf, sem).wait()

        def gat(ib, buf, sem):
            pltpu.async_copy(g_hbm.at[ib], buf, sem)

        def gwait(buf, sem):
            pltpu.make_async_copy(g_hbm.at[ib0], buf, sem).wait()

        def sca(j, buf, sem):
            pltpu.async_copy(buf, acc.at[icv.at[j]], sem, add=True)

        def scawait(buf, sem):
            pltpu.make_async_copy(buf, acc.at[icv.at[0]], sem).wait()

        def scale(buf, eb):
            def srow(i, _2):
                for u in range(4):
                    e = 4 * i + u
                    sv = plsc.load_gather(eb, [jnp.full((16,), e, jnp.int32)])
                    for q in range(nq):
                        buf[e, pl.ds(q * 16, 16)] = buf[e, pl.ds(q * 16, 16)] * sv
                return _2
            lax.fori_loop(0, ch // 4, srow, None)

        si(0, ib0, si0); si(1, ib1, si1)
        se(0, eb0, se0); se(1, eb1, se1)
        siwait(ib0, si0); gat(ib0, g0, sg0)
        siwait(ib1, si1); gat(ib1, g1, sg1)

        def body(j2, _):
            j0 = 2 * j2
            j1 = j0 + 1
            gwait(g0, sg0)
            si(j0 + 2, ib0, si0)
            sewait(eb0, se0)
            scale(g0, eb0)
            se(j0 + 2, eb0, se0)
            sca(j0, g0, ss0)
            gwait(g1, sg1)
            si(j1 + 2, ib1, si1)
            sewait(eb1, se1)
            scale(g1, eb1)
            se(j1 + 2, eb1, se1)
            sca(j1, g1, ss1)
            scawait(g0, ss0); siwait(ib0, si0); gat(ib0, g0, sg0)
            scawait(g1, ss1); siwait(ib1, si1); gat(ib1, g1, sg1)
            return _
        lax.fori_loop(0, cpt // 2 - 1, body, None)

        jl0 = cpt - 2
        gwait(g0, sg0); sewait(eb0, se0); scale(g0, eb0); sca(jl0, g0, ss0)
        gwait(g1, sg1); sewait(eb1, se1); scale(g1, eb1); sca(jl0 + 1, g1, ss1)
        scawait(g0, ss0); scawait(g1, ss1)
        plsc.subcore_barrier()

        def wout(j, _):
            r0 = s * rps + j * ch
            pltpu.sync_copy(acc.at[pl.ds(r0, ch)], g0)
            pltpu.sync_copy(g0, out_hbm.at[c, pl.ds(r0, ch)])
            return _
        lax.fori_loop(0, rps // ch, wout, None)

    ir1a = idxr2 if ir1 is None else ir1
    return k(g, ew2, idxr2, ir1a, idxc2)


def _prep_body(x_ref, emb_ref, out_ref):
    xb = x_ref[...]
    et = xb[:, 1:2].astype(jnp.int32)
    e_row0 = emb_ref[0:1, :]
    e_row1 = emb_ref[1:2, :]
    emb_sel = jnp.where(et == 0, e_row0, e_row1)
    nf = jnp.concatenate([xb[:, 0:1], xb[:, 2:5]], axis=1)
    pad = jnp.zeros((xb.shape[0], 16 - 4 - emb_sel.shape[1]), jnp.float32)
    out_ref[...] = jnp.concatenate([nf, emb_sel, pad], axis=1)


def _prep_table(xp, emb_table, n_pad):
    bn = 1024
    return pl.pallas_call(
        _prep_body,
        grid=(n_pad // bn,),
        in_specs=[
            pl.BlockSpec((bn, 5), lambda i: (i, 0)),
            pl.BlockSpec((2, emb_table.shape[1]), lambda i: (0, 0)),
        ],
        out_specs=pl.BlockSpec((bn, 16), lambda i: (i, 0)),
        out_shape=jax.ShapeDtypeStruct((n_pad, 16), jnp.float32),
    )(xp, emb_table)


def _mlp_body(gcol, grow, ea, w0, w0e, b0, w1, b1, w2, b2, w3, b3, out):
    x32 = jnp.concatenate([gcol[...], grow[...]], axis=1)
    ea_b = ea[...]
    h = jnp.dot(x32, w0[...], preferred_element_type=jnp.float32)
    h = jnp.maximum(h + ea_b * w0e[...] + b0[...], 0.0)
    h = jnp.maximum(jnp.dot(h.astype(jnp.bfloat16), w1[...],
                            preferred_element_type=jnp.float32) + b1[...], 0.0)
    h = jnp.maximum(jnp.dot(h.astype(jnp.bfloat16), w2[...],
                            preferred_element_type=jnp.float32) + b2[...], 0.0)
    m = jnp.dot(h.astype(jnp.bfloat16), w3[...], preferred_element_type=jnp.float32) + b3[...]
    d0 = w3.shape[1] - 32
    col_id = lax.broadcasted_iota(jnp.int32, (1, w3.shape[1]), 1)
    m = m + jnp.where(col_id == d0, ea_b, 0.0)
    w = w3.shape[1] // 2
    out[0] = m[:, :w]
    out[1] = m[:, w:]


def _mlp(gcol, grow, eap, w0cat, w0e, b0, w1, b1, w2, b2, w3a, b3a):
    e_pad = gcol.shape[0]
    hid = w1.shape[0]
    d0a = w3a.shape[1]
    be = 1024
    full = lambda a: pl.BlockSpec(a.shape, lambda i: tuple(0 for _ in a.shape))
    return pl.pallas_call(
        _mlp_body,
        grid=(e_pad // be,),
        in_specs=[
            pl.BlockSpec((be, 16), lambda i: (i, 0)),
            pl.BlockSpec((be, 16), lambda i: (i, 0)),
            pl.BlockSpec((be, 1), lambda i: (i, 0)),
            full(w0cat), full(w0e), full(b0),
            full(w1), full(b1), full(w2), full(b2),
            full(w3a), full(b3a),
        ],
        out_specs=pl.BlockSpec((2, be, d0a // 2), lambda i: (0, i, 0)),
        out_shape=jax.ShapeDtypeStruct((2, e_pad, d0a // 2), jnp.float32),
    )(gcol, grow, eap, w0cat, w0e, b0, w1, b1, w2, b2, w3a, b3a)


def _c_body(h0s, w1p, g1_out, dinv_out):
    a0 = h0s[0]
    a1 = h0s[1]
    w = a0.shape[1]
    d0 = 2 * w - 32
    deg = a1[:, d0 - w:d0 - w + 1]
    dinv = jnp.where(deg > 0, lax.rsqrt(jnp.maximum(deg, 1e-30)), 0.0)
    h0 = jnp.concatenate([a0, a1[:, :d0 - w]], axis=1)
    g1_out[0] = dinv * jnp.dot(h0, w1p[...], preferred_element_type=jnp.float32)
    dinv_out[...] = dinv


def _stage_c(h0split, w1pad):
    _, n_pad, w = h0split.shape
    bn = 1024
    return pl.pallas_call(
        _c_body,
        grid=(2, n_pad // bn),
        in_specs=[
            pl.BlockSpec((2, bn, w), lambda j, i: (0, i, 0)),
            pl.BlockSpec((w1pad.shape[0], 128), lambda j, i: (0, j)),
        ],
        out_specs=[
            pl.BlockSpec((1, bn, 128), lambda j, i: (j, i, 0)),
            pl.BlockSpec((bn, 1), lambda j, i: (i, 0)),
        ],
        out_shape=[
            jax.ShapeDtypeStruct((2, n_pad, 128), jnp.float32),
            jax.ShapeDtypeStruct((n_pad, 1), jnp.float32),
        ],
    )(h0split, w1pad)


def _d_body(t1c, dinv_ref, b1g, w2g, g2_out):
    d1h = b1g.shape[1] // 2
    t = jnp.concatenate([t1c[0][:, :d1h], t1c[1][:, :d1h]], axis=1)
    dinv = dinv_ref[...]
    h1 = jnp.maximum(dinv * t + b1g[...], 0.0)
    g2_out[...] = dinv * jnp.dot(h1, w2g[...], preferred_element_type=jnp.float32)


def _stage_d(t1c, dinv, gcn_b1, gcn_w2):
    n_pad = t1c.shape[1]
    d2 = gcn_w2.shape[1]
    bn = 1024
    return pl.pallas_call(
        _d_body,
        grid=(n_pad // bn,),
        in_specs=[
            pl.BlockSpec((2, bn, 128), lambda i: (0, i, 0)),
            pl.BlockSpec((bn, 1), lambda i: (i, 0)),
            pl.BlockSpec((1, gcn_b1.shape[0]), lambda i: (0, 0)),
            pl.BlockSpec(gcn_w2.shape, lambda i: (0, 0)),
        ],
        out_specs=pl.BlockSpec((bn, d2), lambda i: (i, 0)),
        out_shape=jax.ShapeDtypeStruct((n_pad, d2), jnp.float32),
    )(t1c, dinv, gcn_b1.reshape(1, -1), gcn_w2)


def _e_body(t2, dinv_ref, b2g, out):
    out[...] = jnp.maximum(dinv_ref[...] * (t2[0] + t2[1]) + b2g[...], 0.0)


def _stage_e(t2, dinv, gcn_b2):
    n_pad = t2.shape[1]
    d2 = t2.shape[2]
    bn = 1024
    return pl.pallas_call(
        _e_body,
        grid=(n_pad // bn,),
        in_specs=[
            pl.BlockSpec((2, bn, d2), lambda i: (0, i, 0)),
            pl.BlockSpec((bn, 1), lambda i: (i, 0)),
            pl.BlockSpec((1, d2), lambda i: (0, 0)),
        ],
        out_specs=pl.BlockSpec((bn, d2), lambda i: (i, 0)),
        out_shape=jax.ShapeDtypeStruct((n_pad, d2), jnp.float32),
    )(t2, dinv, gcn_b2.reshape(1, -1))


def kernel(x, edge_attr, edge_index, emb_table,
           phi_w0, phi_b0, phi_w1, phi_b1, phi_w2, phi_b2, phi_w3, phi_b3,
           gcn_w1, gcn_b1, gcn_w2, gcn_b2):
    n = x.shape[0]
    e = edge_index.shape[1]
    hid = phi_w0.shape[1]
    d0 = phi_w3.shape[1]
    d0a = d0 + 32
    n_pad = _ceil_to(n, 2048)
    e_pad = _ceil_to(e, 4096)

    row = edge_index[0]
    col = edge_index[1]
    ep = e_pad - e
    rowp = jnp.concatenate([row, jnp.zeros((ep,), jnp.int32)])
    colp = jnp.concatenate([col, jnp.full((ep,), n, jnp.int32)])
    eap = jnp.concatenate([edge_attr[:, 0], jnp.zeros((ep,), jnp.float32)])[:, None]

    idxc2 = colp.reshape(e_pad // _CH, _CH)
    idxr2 = rowp.reshape(e_pad // _CH, _CH)
    ew2 = eap[:, 0].reshape(e_pad // _CH, _CH)

    xp = jnp.pad(x, ((0, n_pad - n), (0, 0)))
    nf16 = _prep_table(xp, emb_table, n_pad)

    gcol, grow = _sc_edge_gather(nf16, idxc2, idxr2)

    nemb = emb_table.shape[1]
    nfd = 4 + nemb
    w0cat = jnp.zeros((32, hid), jnp.float32)
    w0cat = w0cat.at[0:nfd].set(phi_w0[0:nfd])
    w0cat = w0cat.at[16:16 + nfd].set(phi_w0[nfd:2 * nfd])
    w0e = phi_w0[2 * nfd:2 * nfd + 1]
    w3a = jnp.pad(phi_w3, ((0, 0), (0, 32)))
    b3a = jnp.pad(phi_b3, (0, 32))[None, :]

    msgaug = _mlp(gcol, grow, eap, w0cat, w0e, phi_b0[None, :],
                  phi_w1.astype(jnp.bfloat16), phi_b1[None, :],
                  phi_w2.astype(jnp.bfloat16), phi_b2[None, :],
                  w3a.astype(jnp.bfloat16), b3a)

    h0aug = _sc_msg_scatter(msgaug, idxc2, n_pad)

    d1 = gcn_w1.shape[1]
    d1h = d1 // 2
    w1pad = jnp.zeros((d0, 256), jnp.float32)
    w1pad = w1pad.at[:, 0:d1h].set(gcn_w1[:, :d1h])
    w1pad = w1pad.at[:, 128:128 + d1h].set(gcn_w1[:, d1h:])

    g1c, dinv = _stage_c(h0aug, w1pad)
    t1c = _sc_gather_scale_scatter(g1c.reshape(2 * n_pad, 128), ew2,
                                   idxr2, idxc2, n_pad,
                                   qmax=_ceil_to(d1h, 16) // 16,
                                   colsplit=True, ir1=idxr2 + n_pad)

    g2 = _stage_d(t1c, dinv, gcn_b1, gcn_w2)

    t2 = _sc_gather_scale_scatter(g2, ew2, idxr2, idxc2, n_pad)

    h2 = _stage_e(t2, dinv, gcn_b2)
    return h2[:n]


# speedup vs baseline: 1.0195x; 1.0195x over previous
import functools

import jax
import jax.numpy as jnp
from jax import lax
from jax.experimental import pallas as pl
from jax.experimental.pallas import tpu as pltpu
from jax.experimental.pallas import tpu_sc as plsc


def _ceil_to(a, b):
    return (a + b - 1) // b * b


_SC_INFO = plsc.get_sparse_core_info()
_NC = _SC_INFO.num_cores
_NS = _SC_INFO.num_subcores
_CH = 128


def _sc_mesh():
    return plsc.VectorSubcoreMesh(core_axis_name="c", subcore_axis_name="s")


def _sc_edge_gather(nf16, idxc2, idxr2, ew2):
    nchunks = idxc2.shape[0]
    e_pad = nchunks * _CH
    cpt = nchunks // (_NC * _NS)

    @functools.partial(
        pl.kernel,
        out_type=[jax.ShapeDtypeStruct((e_pad, 16), jnp.float32),
                  jax.ShapeDtypeStruct((e_pad, 16), jnp.float32)],
        mesh=_sc_mesh(),
        compiler_params=pltpu.CompilerParams(use_tc_tiling_on_sc=False, needs_layout_passes=False),
        scratch_types=[pltpu.VMEM((cpt, _CH), jnp.int32),
                       pltpu.VMEM((cpt, _CH), jnp.int32),
                       pltpu.VMEM((cpt, _CH), jnp.float32),
                       pltpu.VMEM((_CH, 16), jnp.float32),
                       pltpu.VMEM((_CH, 16), jnp.float32),
                       pltpu.VMEM((_CH, 16), jnp.float32),
                       pltpu.VMEM((_CH, 16), jnp.float32)]
                      + [pltpu.SemaphoreType.DMA] * 8,
    )
    def k(nf_hbm, ic_hbm, ir_hbm, ew_hbm, gcol_hbm, grow_hbm, icv, irv, ewv,
          bc0, bc1, br0, br1, sgc0, sgc1, sgr0, sgr1, swc0, swc1, swr0, swr1):
        wid = lax.axis_index("s") * _NC + lax.axis_index("c")
        base = wid * cpt
        pltpu.sync_copy(ic_hbm.at[pl.ds(base, cpt)], icv)
        pltpu.sync_copy(ir_hbm.at[pl.ds(base, cpt)], irv)
        pltpu.sync_copy(ew_hbm.at[pl.ds(base, cpt)], ewv)

        lanes = jnp.arange(16, dtype=jnp.int32)
        col15 = jnp.full((16,), 15, jnp.int32)

        def inject(j, buf):
            for g in range(_CH // 16):
                rows = g * 16 + lanes
                vals = ewv[j, pl.ds(g * 16, 16)]
                plsc.store_scatter(buf, [rows, col15], vals)

        def gat(j, idx, buf, sem):
            pltpu.async_copy(nf_hbm.at[idx.at[j]], buf, sem)

        def gwait(buf, sem):
            pltpu.make_async_copy(nf_hbm.at[icv.at[0]], buf, sem).wait()

        def wr(j, buf, out, sem):
            pltpu.async_copy(buf, out.at[pl.ds((base + j) * _CH, _CH)], sem)

        def wwait(buf, sem):
            pltpu.make_async_copy(buf, gcol_hbm.at[pl.ds(0, _CH)], sem).wait()

        gat(0, icv, bc0, sgc0)
        gat(0, irv, br0, sgr0)
        gat(1, icv, bc1, sgc1)
        gat(1, irv, br1, sgr1)

        def body(j2, _):
            j0 = 2 * j2
            j1 = j0 + 1
            gwait(bc0, sgc0); inject(j0, bc0); wr(j0, bc0, gcol_hbm, swc0)
            gwait(br0, sgr0); wr(j0, br0, grow_hbm, swr0)
            gwait(bc1, sgc1); inject(j1, bc1); wr(j1, bc1, gcol_hbm, swc1)
            gwait(br1, sgr1); wr(j1, br1, grow_hbm, swr1)
            wwait(bc0, swc0); gat(j0 + 2, icv, bc0, sgc0)
            wwait(br0, swr0); gat(j0 + 2, irv, br0, sgr0)
            wwait(bc1, swc1); gat(j1 + 2, icv, bc1, sgc1)
            wwait(br1, swr1); gat(j1 + 2, irv, br1, sgr1)
            return _

        lax.fori_loop(0, cpt // 2 - 1, body, None)

        jl0 = cpt - 2
        jl1 = cpt - 1
        gwait(bc0, sgc0); inject(jl0, bc0); wr(jl0, bc0, gcol_hbm, swc0)
        gwait(br0, sgr0); wr(jl0, br0, grow_hbm, swr0)
        gwait(bc1, sgc1); inject(jl1, bc1); wr(jl1, bc1, gcol_hbm, swc1)
        gwait(br1, sgr1); wr(jl1, br1, grow_hbm, swr1)
        wwait(bc0, swc0); wwait(br0, swr0); wwait(bc1, swc1); wwait(br1, swr1)

    return k(nf16, idxc2, idxr2, ew2)


def _sc_msg_scatter(msgsplit, idxc2, n_pad):
    nchunks, _ = idxc2.shape
    w = msgsplit.shape[2]
    rps = n_pad // _NS

    ch = 64
    nch = nchunks * (_CH // ch)
    cpt = nch // _NS

    @functools.partial(
        pl.kernel,
        out_type=jax.ShapeDtypeStruct((_NC, n_pad, w), jnp.float32),
        mesh=_sc_mesh(),
        compiler_params=pltpu.CompilerParams(use_tc_tiling_on_sc=False, needs_layout_passes=False),
        scratch_types=[pltpu.VMEM((ch,), jnp.int32),
                       pltpu.VMEM((ch,), jnp.int32),
                       pltpu.VMEM((ch, w), jnp.float32),
                       pltpu.VMEM((ch, w), jnp.float32),
                       pltpu.VMEM_SHARED((n_pad, w), jnp.float32)]
                      + [pltpu.SemaphoreType.DMA] * 6,
    )
    def k(msg_hbm, ic_hbm, out_hbm, ib0, ib1, m0, m1, acc,
          sr0, sr1, ss0, ss1, si0, si1):
        c = lax.axis_index("c")
        s = lax.axis_index("s")

        def zrow(e, _):
            for q in range(w // 16):
                m0[e, pl.ds(q * 16, 16)] = jnp.zeros((16,), jnp.float32)
            return _
        lax.fori_loop(0, ch, zrow, None)

        def zcp(j, _):
            pltpu.sync_copy(m0, acc.at[pl.ds(s * rps + j * ch, ch)])
            return _
        lax.fori_loop(0, rps // ch, zcp, None)
        plsc.subcore_barrier()

        def si(j, buf, sem):
            pltpu.async_copy(ic_hbm.at[s * cpt + j], buf, sem)

        def siwait(buf, sem):
            pltpu.make_async_copy(ic_hbm.at[0], buf, sem).wait()

        def rd(j, buf, sem):
            e0 = (s * cpt + j) * ch
            pltpu.async_copy(msg_hbm.at[c, pl.ds(e0, ch)], buf, sem)

        def rdwait(buf, sem):
            pltpu.make_async_copy(msg_hbm.at[0, pl.ds(0, ch)], buf, sem).wait()

        def sca(ib, buf, sem):
            pltpu.async_copy(buf, acc.at[ib], sem, add=True)

        def scawait(buf, sem):
            pltpu.make_async_copy(buf, acc.at[ib0], sem).wait()

        si(0, ib0, si0); si(1, ib1, si1)
        rd(0, m0, sr0); rd(1, m1, sr1)

        def body(j2, _):
            j0 = 2 * j2
            j1 = j0 + 1
            rdwait(m0, sr0); siwait(ib0, si0); sca(ib0, m0, ss0)
            rdwait(m1, sr1); siwait(ib1, si1); sca(ib1, m1, ss1)
            scawait(m0, ss0); si(j0 + 2, ib0, si0); rd(j0 + 2, m0, sr0)
            scawait(m1, ss1); si(j1 + 2, ib1, si1); rd(j1 + 2, m1, sr1)
            return _
        lax.fori_loop(0, cpt // 2 - 1, body, None)

        rdwait(m0, sr0); siwait(ib0, si0); sca(ib0, m0, ss0)
        rdwait(m1, sr1); siwait(ib1, si1); sca(ib1, m1, ss1)
        scawait(m0, ss0); scawait(m1, ss1)
        plsc.subcore_barrier()

        def wout(j, _):
            r0 = s * rps + j * ch
            pltpu.sync_copy(acc.at[pl.ds(r0, ch)], m0)
            pltpu.sync_copy(m0, out_hbm.at[c, pl.ds(r0, ch)])
            return _
        lax.fori_loop(0, rps // ch, wout, None)

    return k(msgsplit, idxc2.reshape(nch, ch))


def _sc_gather_scale_scatter(g, ew2, idxr2, idxc2, n_pad, qmax=None,
                             colsplit=False, ir1=None):
    wd = g.shape[1]
    ch = _CH
    nch = idxc2.shape[0]
    cpt = nch // _NS if colsplit else nch // (_NC * _NS)
    rps = n_pad // _NS
    nq = (wd // 16) if qmax is None else qmax

    @functools.partial(
        pl.kernel,
        out_type=jax.ShapeDtypeStruct((_NC, n_pad, wd), jnp.float32),
        mesh=_sc_mesh(),
        compiler_params=pltpu.CompilerParams(use_tc_tiling_on_sc=False, needs_layout_passes=False),
        scratch_types=[pltpu.VMEM((cpt, ch), jnp.int32),
                       pltpu.VMEM((ch,), jnp.int32),
                       pltpu.VMEM((ch,), jnp.int32),
                       pltpu.VMEM((ch,), jnp.float32),
                       pltpu.VMEM((ch,), jnp.float32),
                       pltpu.VMEM((ch, wd), jnp.float32),
                       pltpu.VMEM((ch, wd), jnp.float32),
                       pltpu.VMEM_SHARED((n_pad, wd), jnp.float32)]
                      + [pltpu.SemaphoreType.DMA] * 8,
    )
    def k(g_hbm, ew_hbm, ir0_hbm, ir1_hbm, ic_hbm, out_hbm,
          icv, ib0, ib1, eb0, eb1, g0, g1, acc,
          sg0, sg1, ss0, ss1, si0, si1, se0, se1):
        c = lax.axis_index("c")
        s = lax.axis_index("s")
        base = s * cpt if colsplit else (s * _NC + c) * cpt

        def zrow(e, _):
            for q in range(wd // 16):
                g0[e, pl.ds(q * 16, 16)] = jnp.zeros((16,), jnp.float32)
            return _
        lax.fori_loop(0, ch, zrow, None)

        def zcp(j, _):
            pltpu.sync_copy(g0, acc.at[pl.ds(s * rps + j * ch, ch)])
            return _
        lax.fori_loop(0, rps // ch, zcp, None)
        plsc.subcore_barrier()

        pltpu.sync_copy(ic_hbm.at[pl.ds(base, cpt)], icv)

        if colsplit:
            def si(j, buf, sem):
                @pl.when(c == 0)
                def _a():
                    pltpu.async_copy(ir0_hbm.at[base + j], buf, sem)

                @pl.when(c != 0)
                def _b():
                    pltpu.async_copy(ir1_hbm.at[base + j], buf, sem)
        else:
            def si(j, buf, sem):
                pltpu.async_copy(ir0_hbm.at[base + j], buf, sem)

        def siwait(buf, sem):
            pltpu.make_async_copy(ir0_hbm.at[0], buf, sem).wait()

        def se(j, buf, sem):
            pltpu.async_copy(ew_hbm.at[base + j], buf, sem)

        def sewait(buf, sem):
            pltpu.make_async_copy(ew_hbm.at[0], buf, sem).wait()

        def gat(ib, buf, sem):
            pltpu.async_copy(g_hbm.at[ib], buf, sem)

        def gwait(buf, sem):
            pltpu.make_async_copy(g_hbm.at[ib0], buf, sem).wait()

        def sca(j, buf, sem):
            pltpu.async_copy(buf, acc.at[icv.at[j]], sem, add=True)

        def scawait(buf, sem):
            pltpu.make_async_copy(buf, acc.at[icv.at[0]], sem).wait()

        def scale(buf, eb):
            def srow(i, _2):
                for u in range(4):
                    e = 4 * i + u
                    sv = plsc.load_gather(eb, [jnp.full((16,), e, jnp.int32)])
                    for q in range(nq):
                        buf[e, pl.ds(q * 16, 16)] = buf[e, pl.ds(q * 16, 16)] * sv
                return _2
            lax.fori_loop(0, ch // 4, srow, None)

        si(0, ib0, si0); si(1, ib1, si1)
        se(0, eb0, se0); se(1, eb1, se1)
        siwait(ib0, si0); gat(ib0, g0, sg0)
        siwait(ib1, si1); gat(ib1, g1, sg1)

        def body(j2, _):
            j0 = 2 * j2
            j1 = j0 + 1
            gwait(g0, sg0)
            si(j0 + 2, ib0, si0)
            sewait(eb0, se0)
            scale(g0, eb0)
            se(j0 + 2, eb0, se0)
            sca(j0, g0, ss0)
            gwait(g1, sg1)
            si(j1 + 2, ib1, si1)
            sewait(eb1, se1)
            scale(g1, eb1)
            se(j1 + 2, eb1, se1)
            sca(j1, g1, ss1)
            scawait(g0, ss0); siwait(ib0, si0); gat(ib0, g0, sg0)
            scawait(g1, ss1); siwait(ib1, si1); gat(ib1, g1, sg1)
            return _
        lax.fori_loop(0, cpt // 2 - 1, body, None)

        jl0 = cpt - 2
        gwait(g0, sg0); sewait(eb0, se0); scale(g0, eb0); sca(jl0, g0, ss0)
        gwait(g1, sg1); sewait(eb1, se1); scale(g1, eb1); sca(jl0 + 1, g1, ss1)
        scawait(g0, ss0); scawait(g1, ss1)
        plsc.subcore_barrier()

        def wout(j, _):
            r0 = s * rps + j * ch
            pltpu.sync_copy(acc.at[pl.ds(r0, ch)], g0)
            pltpu.sync_copy(g0, out_hbm.at[c, pl.ds(r0, ch)])
            return _
        lax.fori_loop(0, rps // ch, wout, None)

    ir1a = idxr2 if ir1 is None else ir1
    return k(g, ew2, idxr2, ir1a, idxc2)


def _prep_body(x_ref, emb_ref, out_ref):
    xb = x_ref[...]
    et = xb[:, 1:2].astype(jnp.int32)
    e_row0 = emb_ref[0:1, :]
    e_row1 = emb_ref[1:2, :]
    emb_sel = jnp.where(et == 0, e_row0, e_row1)
    nf = jnp.concatenate([xb[:, 0:1], xb[:, 2:5]], axis=1)
    pad = jnp.zeros((xb.shape[0], 16 - 4 - emb_sel.shape[1]), jnp.float32)
    out_ref[...] = jnp.concatenate([nf, emb_sel, pad], axis=1)


def _prep_table(xp, emb_table, n_pad):
    bn = 1024
    return pl.pallas_call(
        _prep_body,
        grid=(n_pad // bn,),
        in_specs=[
            pl.BlockSpec((bn, 5), lambda i: (i, 0)),
            pl.BlockSpec((2, emb_table.shape[1]), lambda i: (0, 0)),
        ],
        out_specs=pl.BlockSpec((bn, 16), lambda i: (i, 0)),
        out_shape=jax.ShapeDtypeStruct((n_pad, 16), jnp.float32),
    )(xp, emb_table)


def _mlp_body(gcol, grow, ea, w0, b0, w1, b1, w2, b2, w3, b3, out):
    x32 = jnp.concatenate([gcol[...], grow[...]], axis=1)
    ea_b = ea[...]
    h = jnp.dot(x32, w0[...], preferred_element_type=jnp.float32)
    h = jnp.maximum(h + b0[...], 0.0).astype(jnp.bfloat16)
    h = jnp.maximum(jnp.dot(h, w1[...], preferred_element_type=jnp.float32)
                    + b1[...], 0.0).astype(jnp.bfloat16)
    h = jnp.maximum(jnp.dot(h, w2[...], preferred_element_type=jnp.float32)
                    + b2[...], 0.0).astype(jnp.bfloat16)
    m = jnp.dot(h, w3[...], preferred_element_type=jnp.float32) + b3[...]
    d0 = w3.shape[1] - 32
    col_id = lax.broadcasted_iota(jnp.int32, (1, w3.shape[1]), 1)
    m = m + jnp.where(col_id == d0, ea_b, 0.0)
    w = w3.shape[1] // 2
    out[0] = m[:, :w]
    out[1] = m[:, w:]


def _mlp(gcol, grow, eap, w0cat, b0, w1, b1, w2, b2, w3a, b3a):
    e_pad = gcol.shape[0]
    d0a = w3a.shape[1]
    be = 1024
    full = lambda a: pl.BlockSpec(a.shape, lambda i: tuple(0 for _ in a.shape))
    return pl.pallas_call(
        _mlp_body,
        grid=(e_pad // be,),
        in_specs=[
            pl.BlockSpec((be, 16), lambda i: (i, 0)),
            pl.BlockSpec((be, 16), lambda i: (i, 0)),
            pl.BlockSpec((be, 1), lambda i: (i, 0)),
            full(w0cat), full(b0),
            full(w1), full(b1), full(w2), full(b2),
            full(w3a), full(b3a),
        ],
        out_specs=pl.BlockSpec((2, be, d0a // 2), lambda i: (0, i, 0)),
        out_shape=jax.ShapeDtypeStruct((2, e_pad, d0a // 2), jnp.float32),
    )(gcol, grow, eap, w0cat, b0, w1, b1, w2, b2, w3a, b3a)


def _c_body(h0s, w1p, g1_out, dinv_out):
    a0 = h0s[0]
    a1 = h0s[1]
    w = a0.shape[1]
    d0 = 2 * w - 32
    deg = a1[:, d0 - w:d0 - w + 1]
    dinv = jnp.where(deg > 0, lax.rsqrt(jnp.maximum(deg, 1e-30)), 0.0)
    h0 = jnp.concatenate([a0, a1[:, :d0 - w]], axis=1)
    g1_out[0] = dinv * jnp.dot(h0, w1p[...], preferred_element_type=jnp.float32)
    dinv_out[...] = dinv


def _stage_c(h0split, w1pad):
    _, n_pad, w = h0split.shape
    bn = 1024
    return pl.pallas_call(
        _c_body,
        grid=(2, n_pad // bn),
        in_specs=[
            pl.BlockSpec((2, bn, w), lambda j, i: (0, i, 0)),
            pl.BlockSpec((w1pad.shape[0], 128), lambda j, i: (0, j)),
        ],
        out_specs=[
            pl.BlockSpec((1, bn, 128), lambda j, i: (j, i, 0)),
            pl.BlockSpec((bn, 1), lambda j, i: (i, 0)),
        ],
        out_shape=[
            jax.ShapeDtypeStruct((2, n_pad, 128), jnp.float32),
            jax.ShapeDtypeStruct((n_pad, 1), jnp.float32),
        ],
    )(h0split, w1pad)


def _d_body(t1c, dinv_ref, b1g, w2g, g2_out):
    d1h = b1g.shape[1] // 2
    t = jnp.concatenate([t1c[0][:, :d1h], t1c[1][:, :d1h]], axis=1)
    dinv = dinv_ref[...]
    h1 = jnp.maximum(dinv * t + b1g[...], 0.0)
    g2_out[...] = dinv * jnp.dot(h1, w2g[...], preferred_element_type=jnp.float32)


def _stage_d(t1c, dinv, gcn_b1, gcn_w2):
    n_pad = t1c.shape[1]
    d2 = gcn_w2.shape[1]
    bn = 1024
    return pl.pallas_call(
        _d_body,
        grid=(n_pad // bn,),
        in_specs=[
            pl.BlockSpec((2, bn, 128), lambda i: (0, i, 0)),
            pl.BlockSpec((bn, 1), lambda i: (i, 0)),
            pl.BlockSpec((1, gcn_b1.shape[0]), lambda i: (0, 0)),
            pl.BlockSpec(gcn_w2.shape, lambda i: (0, 0)),
        ],
        out_specs=pl.BlockSpec((bn, d2), lambda i: (i, 0)),
        out_shape=jax.ShapeDtypeStruct((n_pad, d2), jnp.float32),
    )(t1c, dinv, gcn_b1.reshape(1, -1), gcn_w2)


def _e_body(t2, dinv_ref, b2g, out):
    out[...] = jnp.maximum(dinv_ref[...] * (t2[0] + t2[1]) + b2g[...], 0.0)


def _stage_e(t2, dinv, gcn_b2):
    n_pad = t2.shape[1]
    d2 = t2.shape[2]
    bn = 1024
    return pl.pallas_call(
        _e_body,
        grid=(n_pad // bn,),
        in_specs=[
            pl.BlockSpec((2, bn, d2), lambda i: (0, i, 0)),
            pl.BlockSpec((bn, 1), lambda i: (i, 0)),
            pl.BlockSpec((1, d2), lambda i: (0, 0)),
        ],
        out_specs=pl.BlockSpec((bn, d2), lambda i: (i, 0)),
        out_shape=jax.ShapeDtypeStruct((n_pad, d2), jnp.float32),
    )(t2, dinv, gcn_b2.reshape(1, -1))


def kernel(x, edge_attr, edge_index, emb_table,
           phi_w0, phi_b0, phi_w1, phi_b1, phi_w2, phi_b2, phi_w3, phi_b3,
           gcn_w1, gcn_b1, gcn_w2, gcn_b2):
    n = x.shape[0]
    e = edge_index.shape[1]
    hid = phi_w0.shape[1]
    d0 = phi_w3.shape[1]
    d0a = d0 + 32
    n_pad = _ceil_to(n, 2048)
    e_pad = _ceil_to(e, 4096)

    row = edge_index[0]
    col = edge_index[1]
    ep = e_pad - e
    rowp = jnp.concatenate([row, jnp.zeros((ep,), jnp.int32)])
    colp = jnp.concatenate([col, jnp.full((ep,), n, jnp.int32)])
    eap = jnp.concatenate([edge_attr[:, 0], jnp.zeros((ep,), jnp.float32)])[:, None]

    idxc2 = colp.reshape(e_pad // _CH, _CH)
    idxr2 = rowp.reshape(e_pad // _CH, _CH)
    ew2 = eap[:, 0].reshape(e_pad // _CH, _CH)

    xp = jnp.pad(x, ((0, n_pad - n), (0, 0)))
    nf16 = _prep_table(xp, emb_table, n_pad)

    gcol, grow = _sc_edge_gather(nf16, idxc2, idxr2, ew2)

    nemb = emb_table.shape[1]
    nfd = 4 + nemb
    w0cat = jnp.zeros((32, hid), jnp.float32)
    w0cat = w0cat.at[0:nfd].set(phi_w0[0:nfd])
    w0cat = w0cat.at[15].set(phi_w0[2 * nfd])
    w0cat = w0cat.at[16:16 + nfd].set(phi_w0[nfd:2 * nfd])
    w3a = jnp.pad(phi_w3, ((0, 0), (0, 32)))
    b3a = jnp.pad(phi_b3, (0, 32))[None, :]

    msgaug = _mlp(gcol, grow, eap, w0cat, phi_b0[None, :],
                  phi_w1.astype(jnp.bfloat16), phi_b1[None, :].astype(jnp.bfloat16),
                  phi_w2.astype(jnp.bfloat16), phi_b2[None, :].astype(jnp.bfloat16),
                  w3a.astype(jnp.bfloat16), b3a)

    h0aug = _sc_msg_scatter(msgaug, idxc2, n_pad)

    d1 = gcn_w1.shape[1]
    d1h = d1 // 2
    w1pad = jnp.zeros((d0, 256), jnp.float32)
    w1pad = w1pad.at[:, 0:d1h].set(gcn_w1[:, :d1h])
    w1pad = w1pad.at[:, 128:128 + d1h].set(gcn_w1[:, d1h:])

    g1c, dinv = _stage_c(h0aug, w1pad)
    t1c = _sc_gather_scale_scatter(g1c.reshape(2 * n_pad, 128), ew2,
                                   idxr2, idxc2, n_pad,
                                   qmax=_ceil_to(d1h, 16) // 16,
                                   colsplit=True, ir1=idxr2 + n_pad)

    g2 = _stage_d(t1c, dinv, gcn_b1, gcn_w2)

    t2 = _sc_gather_scale_scatter(g2, ew2, idxr2, idxc2, n_pad)

    h2 = _stage_e(t2, dinv, gcn_b2)
    return h2[:n]


# speedup vs baseline: 1.1831x; 1.1605x over previous
import functools

import jax
import jax.numpy as jnp
from jax import lax
from jax.experimental import pallas as pl
from jax.experimental.pallas import tpu as pltpu
from jax.experimental.pallas import tpu_sc as plsc


def _ceil_to(a, b):
    return (a + b - 1) // b * b


_SC_INFO = plsc.get_sparse_core_info()
_NC = _SC_INFO.num_cores
_NS = _SC_INFO.num_subcores
_CH = 128


def _sc_mesh():
    return plsc.VectorSubcoreMesh(core_axis_name="c", subcore_axis_name="s")


def _sc_edge_gather(nf16, idxc2, idxr2, ew2):
    nchunks = idxc2.shape[0]
    e_pad = nchunks * _CH
    cpt = nchunks // (_NC * _NS)

    @functools.partial(
        pl.kernel,
        out_type=[jax.ShapeDtypeStruct((e_pad, 16), jnp.float32),
                  jax.ShapeDtypeStruct((e_pad, 16), jnp.float32)],
        mesh=_sc_mesh(),
        compiler_params=pltpu.CompilerParams(use_tc_tiling_on_sc=False, needs_layout_passes=False),
        scratch_types=[pltpu.VMEM((cpt, _CH), jnp.int32),
                       pltpu.VMEM((cpt, _CH), jnp.int32),
                       pltpu.VMEM((cpt, _CH), jnp.float32),
                       pltpu.VMEM((_CH, 16), jnp.float32),
                       pltpu.VMEM((_CH, 16), jnp.float32),
                       pltpu.VMEM((_CH, 16), jnp.float32),
                       pltpu.VMEM((_CH, 16), jnp.float32)]
                      + [pltpu.SemaphoreType.DMA] * 8,
    )
    def k(nf_hbm, ic_hbm, ir_hbm, ew_hbm, gcol_hbm, grow_hbm, icv, irv, ewv,
          bc0, bc1, br0, br1, sgc0, sgc1, sgr0, sgr1, swc0, swc1, swr0, swr1):
        wid = lax.axis_index("s") * _NC + lax.axis_index("c")
        base = wid * cpt
        pltpu.sync_copy(ic_hbm.at[pl.ds(base, cpt)], icv)
        pltpu.sync_copy(ir_hbm.at[pl.ds(base, cpt)], irv)
        pltpu.sync_copy(ew_hbm.at[pl.ds(base, cpt)], ewv)

        lanes = jnp.arange(16, dtype=jnp.int32)
        col15 = jnp.full((16,), 15, jnp.int32)

        def inject(j, buf):
            for g in range(_CH // 16):
                rows = g * 16 + lanes
                vals = ewv[j, pl.ds(g * 16, 16)]
                plsc.store_scatter(buf, [rows, col15], vals)

        def gat(j, idx, buf, sem):
            pltpu.async_copy(nf_hbm.at[idx.at[j]], buf, sem)

        def gwait(buf, sem):
            pltpu.make_async_copy(nf_hbm.at[icv.at[0]], buf, sem).wait()

        def wr(j, buf, out, sem):
            pltpu.async_copy(buf, out.at[pl.ds((base + j) * _CH, _CH)], sem)

        def wwait(buf, sem):
            pltpu.make_async_copy(buf, gcol_hbm.at[pl.ds(0, _CH)], sem).wait()

        gat(0, icv, bc0, sgc0)
        gat(0, irv, br0, sgr0)
        gat(1, icv, bc1, sgc1)
        gat(1, irv, br1, sgr1)

        def body(j2, _):
            j0 = 2 * j2
            j1 = j0 + 1
            gwait(bc0, sgc0); inject(j0, bc0); wr(j0, bc0, gcol_hbm, swc0)
            gwait(br0, sgr0); wr(j0, br0, grow_hbm, swr0)
            gwait(bc1, sgc1); inject(j1, bc1); wr(j1, bc1, gcol_hbm, swc1)
            gwait(br1, sgr1); wr(j1, br1, grow_hbm, swr1)
            wwait(bc0, swc0); gat(j0 + 2, icv, bc0, sgc0)
            wwait(br0, swr0); gat(j0 + 2, irv, br0, sgr0)
            wwait(bc1, swc1); gat(j1 + 2, icv, bc1, sgc1)
            wwait(br1, swr1); gat(j1 + 2, irv, br1, sgr1)
            return _

        lax.fori_loop(0, cpt // 2 - 1, body, None)

        jl0 = cpt - 2
        jl1 = cpt - 1
        gwait(bc0, sgc0); inject(jl0, bc0); wr(jl0, bc0, gcol_hbm, swc0)
        gwait(br0, sgr0); wr(jl0, br0, grow_hbm, swr0)
        gwait(bc1, sgc1); inject(jl1, bc1); wr(jl1, bc1, gcol_hbm, swc1)
        gwait(br1, sgr1); wr(jl1, br1, grow_hbm, swr1)
        wwait(bc0, swc0); wwait(br0, swr0); wwait(bc1, swc1); wwait(br1, swr1)

    return k(nf16, idxc2, idxr2, ew2)


def _sc_msg_scatter(msgsplit, idxc2, n_pad):
    nchunks, _ = idxc2.shape
    w = msgsplit.shape[2]
    rps = n_pad // _NS

    ch = 64
    nch = nchunks * (_CH // ch)
    cpt = nch // _NS

    @functools.partial(
        pl.kernel,
        out_type=jax.ShapeDtypeStruct((_NC, n_pad, w), jnp.float32),
        mesh=_sc_mesh(),
        compiler_params=pltpu.CompilerParams(use_tc_tiling_on_sc=False, needs_layout_passes=False),
        scratch_types=[pltpu.VMEM((ch,), jnp.int32),
                       pltpu.VMEM((ch,), jnp.int32),
                       pltpu.VMEM((ch, w), jnp.float32),
                       pltpu.VMEM((ch, w), jnp.float32),
                       pltpu.VMEM_SHARED((n_pad, w), jnp.float32)]
                      + [pltpu.SemaphoreType.DMA] * 6,
    )
    def k(msg_hbm, ic_hbm, out_hbm, ib0, ib1, m0, m1, acc,
          sr0, sr1, ss0, ss1, si0, si1):
        c = lax.axis_index("c")
        s = lax.axis_index("s")

        def zrow(e, _):
            for q in range(w // 16):
                m0[e, pl.ds(q * 16, 16)] = jnp.zeros((16,), jnp.float32)
            return _
        lax.fori_loop(0, ch, zrow, None)

        def zcp(j, _):
            pltpu.sync_copy(m0, acc.at[pl.ds(s * rps + j * ch, ch)])
            return _
        lax.fori_loop(0, rps // ch, zcp, None)
        plsc.subcore_barrier()

        def si(j, buf, sem):
            pltpu.async_copy(ic_hbm.at[s * cpt + j], buf, sem)

        def siwait(buf, sem):
            pltpu.make_async_copy(ic_hbm.at[0], buf, sem).wait()

        def rd(j, buf, sem):
            e0 = (s * cpt + j) * ch
            pltpu.async_copy(msg_hbm.at[c, pl.ds(e0, ch)], buf, sem)

        def rdwait(buf, sem):
            pltpu.make_async_copy(msg_hbm.at[0, pl.ds(0, ch)], buf, sem).wait()

        def sca(ib, buf, sem):
            pltpu.async_copy(buf, acc.at[ib], sem, add=True)

        def scawait(buf, sem):
            pltpu.make_async_copy(buf, acc.at[ib0], sem).wait()

        si(0, ib0, si0); si(1, ib1, si1)
        rd(0, m0, sr0); rd(1, m1, sr1)

        def body(j2, _):
            j0 = 2 * j2
            j1 = j0 + 1
            rdwait(m0, sr0); siwait(ib0, si0); sca(ib0, m0, ss0)
            rdwait(m1, sr1); siwait(ib1, si1); sca(ib1, m1, ss1)
            scawait(m0, ss0); si(j0 + 2, ib0, si0); rd(j0 + 2, m0, sr0)
            scawait(m1, ss1); si(j1 + 2, ib1, si1); rd(j1 + 2, m1, sr1)
            return _
        lax.fori_loop(0, cpt // 2 - 1, body, None)

        rdwait(m0, sr0); siwait(ib0, si0); sca(ib0, m0, ss0)
        rdwait(m1, sr1); siwait(ib1, si1); sca(ib1, m1, ss1)
        scawait(m0, ss0); scawait(m1, ss1)
        plsc.subcore_barrier()

        def wout(j, _):
            r0 = s * rps + j * ch
            pltpu.sync_copy(acc.at[pl.ds(r0, ch)], m0)
            pltpu.sync_copy(m0, out_hbm.at[c, pl.ds(r0, ch)])
            return _
        lax.fori_loop(0, rps // ch, wout, None)

    return k(msgsplit, idxc2.reshape(nch, ch))


def _sc_gather_scale_scatter(g, ew2, idxr2, idxc2, n_pad, qmax=None,
                             colsplit=False, ir1=None):
    wd = g.shape[1]
    ch = _CH
    nch = idxc2.shape[0]
    cpt = nch // _NS if colsplit else nch // (_NC * _NS)
    rps = n_pad // _NS
    nq = (wd // 16) if qmax is None else qmax

    @functools.partial(
        pl.kernel,
        out_type=jax.ShapeDtypeStruct((_NC, n_pad, wd), jnp.float32),
        mesh=_sc_mesh(),
        compiler_params=pltpu.CompilerParams(use_tc_tiling_on_sc=False, needs_layout_passes=False),
        scratch_types=[pltpu.VMEM((cpt, ch), jnp.int32),
                       pltpu.VMEM((ch,), jnp.int32),
                       pltpu.VMEM((ch,), jnp.int32),
                       pltpu.VMEM((ch,), jnp.float32),
                       pltpu.VMEM((ch,), jnp.float32),
                       pltpu.VMEM((ch, wd), jnp.float32),
                       pltpu.VMEM((ch, wd), jnp.float32),
                       pltpu.VMEM_SHARED((n_pad, wd), jnp.float32)]
                      + [pltpu.SemaphoreType.DMA] * 8,
    )
    def k(g_hbm, ew_hbm, ir0_hbm, ir1_hbm, ic_hbm, out_hbm,
          icv, ib0, ib1, eb0, eb1, g0, g1, acc,
          sg0, sg1, ss0, ss1, si0, si1, se0, se1):
        c = lax.axis_index("c")
        s = lax.axis_index("s")
        base = s * cpt if colsplit else (s * _NC + c) * cpt

        def zrow(e, _):
            for q in range(wd // 16):
                g0[e, pl.ds(q * 16, 16)] = jnp.zeros((16,), jnp.float32)
            return _
        lax.fori_loop(0, ch, zrow, None)

        def zcp(j, _):
            pltpu.sync_copy(g0, acc.at[pl.ds(s * rps + j * ch, ch)])
            return _
        lax.fori_loop(0, rps // ch, zcp, None)
        plsc.subcore_barrier()

        pltpu.sync_copy(ic_hbm.at[pl.ds(base, cpt)], icv)

        if colsplit:
            def si(j, buf, sem):
                @pl.when(c == 0)
                def _a():
                    pltpu.async_copy(ir0_hbm.at[base + j], buf, sem)

                @pl.when(c != 0)
                def _b():
                    pltpu.async_copy(ir1_hbm.at[base + j], buf, sem)
        else:
            def si(j, buf, sem):
                pltpu.async_copy(ir0_hbm.at[base + j], buf, sem)

        def siwait(buf, sem):
            pltpu.make_async_copy(ir0_hbm.at[0], buf, sem).wait()

        def se(j, buf, sem):
            pltpu.async_copy(ew_hbm.at[base + j], buf, sem)

        def sewait(buf, sem):
            pltpu.make_async_copy(ew_hbm.at[0], buf, sem).wait()

        def gat(ib, buf, sem):
            pltpu.async_copy(g_hbm.at[ib], buf, sem)

        def gwait(buf, sem):
            pltpu.make_async_copy(g_hbm.at[ib0], buf, sem).wait()

        def sca(j, buf, sem):
            pltpu.async_copy(buf, acc.at[icv.at[j]], sem, add=True)

        def scawait(buf, sem):
            pltpu.make_async_copy(buf, acc.at[icv.at[0]], sem).wait()

        def scale(buf, eb):
            def srow(i, _2):
                for u in range(4):
                    e = 4 * i + u
                    sv = plsc.load_gather(eb, [jnp.full((16,), e, jnp.int32)])
                    for q in range(nq):
                        buf[e, pl.ds(q * 16, 16)] = buf[e, pl.ds(q * 16, 16)] * sv
                return _2
            lax.fori_loop(0, ch // 4, srow, None)

        si(0, ib0, si0); si(1, ib1, si1)
        se(0, eb0, se0); se(1, eb1, se1)
        siwait(ib0, si0); gat(ib0, g0, sg0)
        siwait(ib1, si1); gat(ib1, g1, sg1)

        def body(j2, _):
            j0 = 2 * j2
            j1 = j0 + 1
            gwait(g0, sg0)
            si(j0 + 2, ib0, si0)
            sewait(eb0, se0)
            scale(g0, eb0)
            se(j0 + 2, eb0, se0)
            sca(j0, g0, ss0)
            gwait(g1, sg1)
            si(j1 + 2, ib1, si1)
            sewait(eb1, se1)
            scale(g1, eb1)
            se(j1 + 2, eb1, se1)
            sca(j1, g1, ss1)
            scawait(g0, ss0); siwait(ib0, si0); gat(ib0, g0, sg0)
            scawait(g1, ss1); siwait(ib1, si1); gat(ib1, g1, sg1)
            return _
        lax.fori_loop(0, cpt // 2 - 1, body, None)

        jl0 = cpt - 2
        gwait(g0, sg0); sewait(eb0, se0); scale(g0, eb0); sca(jl0, g0, ss0)
        gwait(g1, sg1); sewait(eb1, se1); scale(g1, eb1); sca(jl0 + 1, g1, ss1)
        scawait(g0, ss0); scawait(g1, ss1)
        plsc.subcore_barrier()

        def wout(j, _):
            r0 = s * rps + j * ch
            pltpu.sync_copy(acc.at[pl.ds(r0, ch)], g0)
            pltpu.sync_copy(g0, out_hbm.at[c, pl.ds(r0, ch)])
            return _
        lax.fori_loop(0, rps // ch, wout, None)

    ir1a = idxr2 if ir1 is None else ir1
    return k(g, ew2, idxr2, ir1a, idxc2)


def _prep_body(x_ref, emb_ref, out_ref):
    xb = x_ref[...]
    et = xb[:, 1:2].astype(jnp.int32)
    e_row0 = emb_ref[0:1, :]
    e_row1 = emb_ref[1:2, :]
    emb_sel = jnp.where(et == 0, e_row0, e_row1)
    nf = jnp.concatenate([xb[:, 0:1], xb[:, 2:5]], axis=1)
    pad = jnp.zeros((xb.shape[0], 16 - 4 - emb_sel.shape[1]), jnp.float32)
    out_ref[...] = jnp.concatenate([nf, emb_sel, pad], axis=1)


def _prep_table(xp, emb_table, n_pad):
    bn = 1024
    return pl.pallas_call(
        _prep_body,
        grid=(n_pad // bn,),
        in_specs=[
            pl.BlockSpec((bn, 5), lambda i: (i, 0)),
            pl.BlockSpec((2, emb_table.shape[1]), lambda i: (0, 0)),
        ],
        out_specs=pl.BlockSpec((bn, 16), lambda i: (i, 0)),
        out_shape=jax.ShapeDtypeStruct((n_pad, 16), jnp.float32),
    )(xp, emb_table)


def _mlp_body(gcol, grow, ea, w0, b0, w1, b1, w2, b2, w3, b3, out):
    x32 = jnp.concatenate([gcol[...], grow[...]], axis=1)
    ea_b = ea[...]
    h = jnp.dot(x32, w0[...], preferred_element_type=jnp.float32)
    h = jnp.maximum(h + b0[...], 0.0).astype(jnp.bfloat16)
    h = jnp.maximum(jnp.dot(h, w1[...], preferred_element_type=jnp.float32)
                    + b1[...], 0.0).astype(jnp.bfloat16)
    h = jnp.maximum(jnp.dot(h, w2[...], preferred_element_type=jnp.float32)
                    + b2[...], 0.0).astype(jnp.bfloat16)
    m = jnp.dot(h, w3[...], preferred_element_type=jnp.float32) + b3[...]
    d0 = w3.shape[1] - 32
    col_id = lax.broadcasted_iota(jnp.int32, (1, w3.shape[1]), 1)
    m = m + jnp.where(col_id == d0, ea_b, 0.0)
    w = w3.shape[1] // 2
    out[0] = m[:, :w]
    out[1] = m[:, w:]


def _mlp(gcol, grow, eap, w0cat, b0, w1, b1, w2, b2, w3a, b3a):
    e_pad = gcol.shape[0]
    d0a = w3a.shape[1]
    be = 1024
    full = lambda a: pl.BlockSpec(a.shape, lambda i: tuple(0 for _ in a.shape))
    return pl.pallas_call(
        _mlp_body,
        grid=(e_pad // be,),
        in_specs=[
            pl.BlockSpec((be, 16), lambda i: (i, 0)),
            pl.BlockSpec((be, 16), lambda i: (i, 0)),
            pl.BlockSpec((be, 1), lambda i: (i, 0)),
            full(w0cat), full(b0),
            full(w1), full(b1), full(w2), full(b2),
            full(w3a), full(b3a),
        ],
        out_specs=pl.BlockSpec((2, be, d0a // 2), lambda i: (0, i, 0)),
        out_shape=jax.ShapeDtypeStruct((2, e_pad, d0a // 2), jnp.float32),
    )(gcol, grow, eap, w0cat, b0, w1, b1, w2, b2, w3a, b3a)


def _c_body(h0s, w1p, g1_out, dinv_out):
    a0 = h0s[0]
    a1 = h0s[1]
    w = a0.shape[1]
    d0 = 2 * w - 32
    deg = a1[:, d0 - w:d0 - w + 1]
    dinv = jnp.where(deg > 0, lax.rsqrt(jnp.maximum(deg, 1e-30)), 0.0)
    h0 = jnp.concatenate([a0, a1[:, :d0 - w]], axis=1)
    g1_out[0] = dinv * jnp.dot(h0, w1p[...], preferred_element_type=jnp.float32)
    dinv_out[...] = dinv


def _stage_c(h0split, w1pad):
    _, n_pad, w = h0split.shape
    bn = 1024
    return pl.pallas_call(
        _c_body,
        grid=(2, n_pad // bn),
        in_specs=[
            pl.BlockSpec((2, bn, w), lambda j, i: (0, i, 0)),
            pl.BlockSpec((w1pad.shape[0], 128), lambda j, i: (0, j)),
        ],
        out_specs=[
            pl.BlockSpec((1, bn, 128), lambda j, i: (j, i, 0)),
            pl.BlockSpec((bn, 1), lambda j, i: (i, 0)),
        ],
        out_shape=[
            jax.ShapeDtypeStruct((2, n_pad, 128), jnp.float32),
            jax.ShapeDtypeStruct((n_pad, 1), jnp.float32),
        ],
    )(h0split, w1pad)


def _d_body(t1c, dinv_ref, b1g, w2g, g2_out):
    d1h = b1g.shape[1] // 2
    t = jnp.concatenate([t1c[0][:, :d1h], t1c[1][:, :d1h]], axis=1)
    dinv = dinv_ref[...]
    h1 = jnp.maximum(dinv * t + b1g[...], 0.0)
    g2_out[...] = dinv * jnp.dot(h1, w2g[...], preferred_element_type=jnp.float32)


def _stage_d(t1c, dinv, gcn_b1, gcn_w2):
    n_pad = t1c.shape[1]
    d2 = gcn_w2.shape[1]
    bn = 1024
    return pl.pallas_call(
        _d_body,
        grid=(n_pad // bn,),
        in_specs=[
            pl.BlockSpec((2, bn, 128), lambda i: (0, i, 0)),
            pl.BlockSpec((bn, 1), lambda i: (i, 0)),
            pl.BlockSpec((1, gcn_b1.shape[0]), lambda i: (0, 0)),
            pl.BlockSpec(gcn_w2.shape, lambda i: (0, 0)),
        ],
        out_specs=pl.BlockSpec((bn, d2), lambda i: (i, 0)),
        out_shape=jax.ShapeDtypeStruct((n_pad, d2), jnp.float32),
    )(t1c, dinv, gcn_b1.reshape(1, -1), gcn_w2)


def _e_body(t2, dinv_ref, b2g, out):
    out[...] = jnp.maximum(dinv_ref[...] * (t2[0] + t2[1]) + b2g[...], 0.0)


def _stage_e(t2, dinv, gcn_b2):
    n_pad = t2.shape[1]
    d2 = t2.shape[2]
    bn = 1024
    return pl.pallas_call(
        _e_body,
        grid=(n_pad // bn,),
        in_specs=[
            pl.BlockSpec((2, bn, d2), lambda i: (0, i, 0)),
            pl.BlockSpec((bn, 1), lambda i: (i, 0)),
            pl.BlockSpec((1, d2), lambda i: (0, 0)),
        ],
        out_specs=pl.BlockSpec((bn, d2), lambda i: (i, 0)),
        out_shape=jax.ShapeDtypeStruct((n_pad, d2), jnp.float32),
    )(t2, dinv, gcn_b2.reshape(1, -1))


def kernel(x, edge_attr, edge_index, emb_table,
           phi_w0, phi_b0, phi_w1, phi_b1, phi_w2, phi_b2, phi_w3, phi_b3,
           gcn_w1, gcn_b1, gcn_w2, gcn_b2):
    n = x.shape[0]
    e = edge_index.shape[1]
    hid = phi_w0.shape[1]
    d0 = phi_w3.shape[1]
    d0a = d0 + 32
    n_pad = _ceil_to(n, 2048)
    e_pad = _ceil_to(e, 4096)

    row = edge_index[0]
    col = edge_index[1]
    ep = e_pad - e
    spread = jnp.arange(ep, dtype=jnp.int32)
    rowp = jnp.concatenate([row, spread % n])
    colp = jnp.concatenate([col, n + spread % (n_pad - n)])
    eap = jnp.concatenate([edge_attr[:, 0], jnp.zeros((ep,), jnp.float32)])[:, None]

    idxc2 = colp.reshape(e_pad // _CH, _CH)
    idxr2 = rowp.reshape(e_pad // _CH, _CH)
    ew2 = eap[:, 0].reshape(e_pad // _CH, _CH)

    xp = jnp.pad(x, ((0, n_pad - n), (0, 0)))
    nf16 = _prep_table(xp, emb_table, n_pad)

    gcol, grow = _sc_edge_gather(nf16, idxc2, idxr2, ew2)

    nemb = emb_table.shape[1]
    nfd = 4 + nemb
    w0cat = jnp.zeros((32, hid), jnp.float32)
    w0cat = w0cat.at[0:nfd].set(phi_w0[0:nfd])
    w0cat = w0cat.at[15].set(phi_w0[2 * nfd])
    w0cat = w0cat.at[16:16 + nfd].set(phi_w0[nfd:2 * nfd])
    w3a = jnp.pad(phi_w3, ((0, 0), (0, 32)))
    b3a = jnp.pad(phi_b3, (0, 32))[None, :]

    msgaug = _mlp(gcol, grow, eap, w0cat, phi_b0[None, :],
                  phi_w1.astype(jnp.bfloat16), phi_b1[None, :].astype(jnp.bfloat16),
                  phi_w2.astype(jnp.bfloat16), phi_b2[None, :].astype(jnp.bfloat16),
                  w3a.astype(jnp.bfloat16), b3a)

    h0aug = _sc_msg_scatter(msgaug, idxc2, n_pad)

    d1 = gcn_w1.shape[1]
    d1h = d1 // 2
    w1pad = jnp.zeros((d0, 256), jnp.float32)
    w1pad = w1pad.at[:, 0:d1h].set(gcn_w1[:, :d1h])
    w1pad = w1pad.at[:, 128:128 + d1h].set(gcn_w1[:, d1h:])

    g1c, dinv = _stage_c(h0aug, w1pad)
    t1c = _sc_gather_scale_scatter(g1c.reshape(2 * n_pad, 128), ew2,
                                   idxr2, idxc2, n_pad,
                                   qmax=_ceil_to(d1h, 16) // 16,
                                   colsplit=True, ir1=idxr2 + n_pad)

    g2 = _stage_d(t1c, dinv, gcn_b1, gcn_w2)

    t2 = _sc_gather_scale_scatter(g2, ew2, idxr2, idxc2, n_pad)

    h2 = _stage_e(t2, dinv, gcn_b2)
    return h2[:n]
